# baseline (device time: 1184391 ns/iter reference)
import jax
import jax.numpy as jnp
from jax import lax
from jax.experimental import pallas as pl
from jax.experimental.pallas import tpu as pltpu

N_DEV = 32
M = 4096
N_OUT = 2048
CHUNK = M // N_DEV


def kernel(x, w_mat, scale_x, scale_w):
    k_shard = x.shape[1]

    def body(x_ref, w_ref, sx_ref, sw_ref, out_ref,
             xbf, wbf, comm, send_sems, recv_sems, credit_sem):
        me = lax.axis_index("i")
        left = lax.rem(me + N_DEV - 1, N_DEV)
        right = lax.rem(me + 1, N_DEV)

        barrier = pltpu.get_barrier_semaphore()
        for nbr in (left, right):
            pl.semaphore_signal(
                barrier, inc=1,
                device_id=(nbr,), device_id_type=pl.DeviceIdType.MESH,
            )
        pl.semaphore_wait(barrier, 2)

        xbf[...] = x_ref[...].astype(jnp.bfloat16)
        wbf[...] = w_ref[...].astype(jnp.bfloat16)
        scale = sx_ref[0] * sw_ref[0]

        def pchunk(c):
            xa = xbf[pl.ds(c * CHUNK, CHUNK), :]
            return jnp.dot(xa, wbf[...], preferred_element_type=jnp.float32)

        comm[0, :, :] = pchunk(me)

        for t in range(2 * (N_DEV - 1)):
            s_slot = t % 2
            r_slot = (t + 1) % 2

            pl.semaphore_signal(
                credit_sem, inc=1,
                device_id=(left,), device_id_type=pl.DeviceIdType.MESH,
            )
            pl.semaphore_wait(credit_sem, 1)

            rdma = pltpu.make_async_remote_copy(
                src_ref=comm.at[s_slot],
                dst_ref=comm.at[r_slot],
                send_sem=send_sems.at[s_slot],
                recv_sem=recv_sems.at[r_slot],
                device_id=(right,),
                device_id_type=pl.DeviceIdType.MESH,
            )
            rdma.start()
            rdma.wait()

            if t < N_DEV - 1:
                idx = lax.rem(me + 2 * N_DEV - 1 - t, N_DEV)
                comm[r_slot, :, :] = comm[r_slot, :, :] + pchunk(idx)
                if t == N_DEV - 2:
                    own = lax.rem(me + 1, N_DEV)
                    out_ref[pl.ds(own * CHUNK, CHUNK), :] = (
                        comm[r_slot, :, :] * scale
                    )
            else:
                s = t - (N_DEV - 1)
                idx = lax.rem(me + 2 * N_DEV - s, N_DEV)
                out_ref[pl.ds(idx * CHUNK, CHUNK), :] = (
                    comm[r_slot, :, :] * scale
                )

    out_shape = jax.ShapeDtypeStruct((M, N_OUT), jnp.float32)
    return pl.pallas_call(
        body,
        out_shape=out_shape,
        in_specs=[
            pl.BlockSpec(memory_space=pltpu.VMEM),
            pl.BlockSpec(memory_space=pltpu.VMEM),
            pl.BlockSpec(memory_space=pltpu.SMEM),
            pl.BlockSpec(memory_space=pltpu.SMEM),
        ],
        out_specs=pl.BlockSpec(memory_space=pltpu.VMEM),
        scratch_shapes=[
            pltpu.VMEM((M, k_shard), jnp.bfloat16),
            pltpu.VMEM((k_shard, N_OUT), jnp.bfloat16),
            pltpu.VMEM((2, CHUNK, N_OUT), jnp.float32),
            pltpu.SemaphoreType.DMA((2,)),
            pltpu.SemaphoreType.DMA((2,)),
            pltpu.SemaphoreType.REGULAR,
        ],
        compiler_params=pltpu.CompilerParams(
            collective_id=0,
            vmem_limit_bytes=60 * 1024 * 1024,
        ),
    )(x, w_mat, scale_x, scale_w)
